# baseline (speedup 1.0000x reference)
import functools

import jax
import jax.numpy as jnp
from jax import lax
from jax.experimental import pallas as pl
from jax.experimental.pallas import tpu as pltpu
from jax.experimental.pallas import tpu_sc as plsc

F32 = jnp.float32

NC = 2
NS = 16
NW = NC * NS
CHUNK = 128


def _mesh():
    return plsc.VectorSubcoreMesh(core_axis_name="c", subcore_axis_name="s")


def _make_deg_kernel(npad, nchunk):
    rows = npad // NS

    @functools.partial(
        pl.kernel,
        out_type=jax.ShapeDtypeStruct((NC, npad), F32),
        mesh=_mesh(),
        compiler_params=pltpu.CompilerParams(use_tc_tiling_on_sc=False),
        scratch_types=[
            pltpu.VMEM((nchunk, CHUNK), jnp.int32),
            pltpu.VMEM((CHUNK,), F32),
            pltpu.VMEM_SHARED((npad,), F32),
        ],
    )
    def deg_kernel(dst_hbm, ones_hbm, zhist_hbm, deg_hbm, idx_v, ones_v,
                   hist_sp):
        cid = lax.axis_index("c")
        sid = lax.axis_index("s")
        wid = cid * NS + sid
        base = sid * rows
        pltpu.sync_copy(zhist_hbm.at[pl.ds(base, rows)],
                        hist_sp.at[pl.ds(base, rows)])
        pltpu.sync_copy(dst_hbm.at[wid], idx_v)
        pltpu.sync_copy(ones_hbm, ones_v)
        plsc.subcore_barrier()

        def body(j, _):
            pltpu.sync_copy(ones_v, hist_sp.at[idx_v.at[j]], add=True)
            return 0

        lax.fori_loop(0, nchunk, body, 0)
        plsc.subcore_barrier()
        pltpu.sync_copy(hist_sp.at[pl.ds(base, rows)],
                        deg_hbm.at[cid, pl.ds(base, rows)])

    return deg_kernel


GRP = 8


def _make_mp_kernel(npad, h, nchunk):
    rows = npad // NS
    ngrp = nchunk // GRP
    NB = 4

    @functools.partial(
        pl.kernel,
        out_type=jax.ShapeDtypeStruct((NC, npad, h), F32),
        mesh=_mesh(),
        compiler_params=pltpu.CompilerParams(use_tc_tiling_on_sc=False),
        scratch_types=[
            pltpu.VMEM((nchunk, CHUNK), jnp.int32),
            pltpu.VMEM((nchunk, CHUNK), jnp.int32),
            pltpu.VMEM((NB * GRP, CHUNK, h), F32),
            pltpu.VMEM_SHARED((npad, h), F32),
            pltpu.SemaphoreType.DMA((NB,)),
            pltpu.SemaphoreType.DMA((NB,)),
        ],
    )
    def mp_kernel(y_hbm, src_hbm, dst_hbm, zrows_hbm, parts_hbm,
                  src_v, dst_v, bufs, accum, gsem, ssem):
        cid = lax.axis_index("c")
        sid = lax.axis_index("s")
        wid = cid * NS + sid
        base = sid * rows
        pltpu.sync_copy(zrows_hbm, accum.at[pl.ds(base, rows)])
        pltpu.sync_copy(src_hbm.at[wid], src_v)
        pltpu.sync_copy(dst_hbm.at[wid], dst_v)
        plsc.subcore_barrier()

        def fire_gathers(g, b):
            for k in range(GRP):
                pltpu.async_copy(
                    y_hbm.at[src_v.at[g * GRP + k]],
                    bufs.at[b * GRP + k], gsem.at[b])

        def drain(sem, b):
            for k in range(GRP):
                pltpu.make_async_copy(
                    bufs.at[k], accum.at[pl.ds(0, CHUNK)], sem.at[b]).wait()

        fire_gathers(0, 0)

        def body(g, _):
            b = lax.rem(g, NB)
            nb = lax.rem(g + 1, NB)

            @pl.when(jnp.logical_and(g + 1 < ngrp, g + 1 >= NB))
            def _():
                drain(ssem, nb)

            @pl.when(g + 1 < ngrp)
            def _():
                fire_gathers(g + 1, nb)

            drain(gsem, b)
            for k in range(GRP):
                pltpu.async_copy(
                    bufs.at[b * GRP + k],
                    accum.at[dst_v.at[g * GRP + k]], ssem.at[b], add=True)
            return 0

        lax.fori_loop(0, ngrp, body, 0)
        for j in range(max(ngrp - NB, 0), ngrp):
            drain(ssem, j % NB)
        plsc.subcore_barrier()
        pltpu.sync_copy(accum.at[pl.ds(base, rows)],
                        parts_hbm.at[cid, pl.ds(base, rows)])

    return mp_kernel


def _tc1_body(x_ref, w_ref, deg_ref, y_ref, dinv_ref):
    d = deg_ref[0] + deg_ref[1] + 1.0
    dinv = lax.rsqrt(d)
    xw = jnp.dot(x_ref[...], w_ref[...], preferred_element_type=F32)
    y_ref[...] = xw * dinv
    dinv_ref[...] = dinv


def _tc2_body(p_ref, y1_ref, dinv_ref, b1_ref, w2_ref, y2_ref):
    dinv = dinv_ref[...]
    s = p_ref[0] + p_ref[1] + y1_ref[...]
    hh = jnp.maximum(dinv * s + b1_ref[...], 0.0)
    y2_ref[...] = dinv * jnp.dot(hh, w2_ref[...], preferred_element_type=F32)


def _tc3_body(p_ref, y2_ref, dinv_ref, b2_ref, o_ref):
    o = dinv_ref[...] * (p_ref[0] + p_ref[1] + y2_ref[...]) + b2_ref[...]
    m = jnp.max(o, axis=1, keepdims=True)
    e = jnp.exp(o - m)
    s = jnp.sum(e, axis=1, keepdims=True)
    o_ref[...] = o - m - jnp.log(s)


def kernel(x, edge_index, W1, b1, W2, b2):
    n, d_feat = x.shape
    e = edge_index.shape[1]
    h1 = W1.shape[1]
    h2 = W2.shape[1]

    npad = ((n + NS * 16) // (NS * 16)) * (NS * 16)
    nchunk = (-(-e // (NW * CHUNK * GRP))) * GRP
    epad = nchunk * NW * CHUNK
    epw = epad // NW

    pad = epad - e
    src_p = jnp.concatenate([edge_index[0], jnp.zeros((pad,), jnp.int32)])
    dst_p = jnp.concatenate(
        [edge_index[1], jnp.full((pad,), n, jnp.int32)])
    src_r = src_p.reshape(NW, nchunk, CHUNK)
    dst_r = dst_p.reshape(NW, nchunk, CHUNK)
    x_p = jnp.pad(x, ((0, npad - n), (0, 0)))
    zhist = jnp.zeros((npad,), F32)
    zrows = jnp.zeros((npad // NS, h1), F32)
    ones_c = jnp.ones((CHUNK,), F32)

    deg_kernel = _make_deg_kernel(npad, nchunk)
    mp1 = _make_mp_kernel(npad, h1, nchunk)

    degp = deg_kernel(dst_r, ones_c, zhist)

    y1, dinv = pl.pallas_call(
        _tc1_body,
        out_shape=(
            jax.ShapeDtypeStruct((npad, h1), F32),
            jax.ShapeDtypeStruct((npad, 1), F32),
        ),
    )(x_p, W1, degp.reshape(NC, npad, 1))

    p1 = mp1(y1, src_r, dst_r, zrows)

    y2 = pl.pallas_call(
        _tc2_body,
        out_shape=jax.ShapeDtypeStruct((npad, h2), F32),
    )(p1, y1, dinv, b1.reshape(1, h1), W2)

    if h2 != h1:
        mp2 = _make_mp_kernel(npad, h2, nchunk)
        zrows2 = jnp.zeros((npad // NS, h2), F32)
    else:
        mp2, zrows2 = mp1, zrows
    p2 = mp2(y2, src_r, dst_r, zrows2)

    out = pl.pallas_call(
        _tc3_body,
        out_shape=jax.ShapeDtypeStruct((npad, h2), F32),
    )(p2, y2, dinv, b2.reshape(1, h2))

    return out[:n]


# speedup vs baseline: 1.3101x; 1.3101x over previous
import functools

import jax
import jax.numpy as jnp
from jax import lax
from jax.experimental import pallas as pl
from jax.experimental.pallas import tpu as pltpu
from jax.experimental.pallas import tpu_sc as plsc

F32 = jnp.float32

NC = 2
NS = 16
NW = NC * NS
CHUNK = 128


def _mesh():
    return plsc.VectorSubcoreMesh(core_axis_name="c", subcore_axis_name="s")


def _make_deg_kernel(npad, nchunk):
    rows = npad // NS

    @functools.partial(
        pl.kernel,
        out_type=jax.ShapeDtypeStruct((NC, npad), F32),
        mesh=_mesh(),
        compiler_params=pltpu.CompilerParams(use_tc_tiling_on_sc=False),
        scratch_types=[
            pltpu.VMEM((nchunk, CHUNK), jnp.int32),
            pltpu.VMEM((CHUNK,), F32),
            pltpu.VMEM_SHARED((npad,), F32),
        ],
    )
    def deg_kernel(dst_hbm, ones_hbm, zhist_hbm, deg_hbm, idx_v, ones_v,
                   hist_sp):
        cid = lax.axis_index("c")
        sid = lax.axis_index("s")
        wid = cid * NS + sid
        base = sid * rows
        pltpu.sync_copy(zhist_hbm.at[pl.ds(base, rows)],
                        hist_sp.at[pl.ds(base, rows)])
        pltpu.sync_copy(dst_hbm.at[wid], idx_v)
        pltpu.sync_copy(ones_hbm, ones_v)
        plsc.subcore_barrier()

        def body(j, _):
            pltpu.sync_copy(ones_v, hist_sp.at[idx_v.at[j]], add=True)
            return 0

        lax.fori_loop(0, nchunk, body, 0)
        plsc.subcore_barrier()
        pltpu.sync_copy(hist_sp.at[pl.ds(base, rows)],
                        deg_hbm.at[cid, pl.ds(base, rows)])

    return deg_kernel


GRP = 4


def _make_mp_kernel(npad, h, nchunk):
    rows = npad // NS
    ngrp = nchunk // GRP
    NB = 4

    @functools.partial(
        pl.kernel,
        out_type=jax.ShapeDtypeStruct((NC, npad, h), F32),
        mesh=_mesh(),
        compiler_params=pltpu.CompilerParams(use_tc_tiling_on_sc=False),
        scratch_types=[
            pltpu.VMEM((nchunk, CHUNK), jnp.int32),
            pltpu.VMEM((nchunk, CHUNK), jnp.int32),
            pltpu.VMEM((NB * GRP, CHUNK, h), F32),
            pltpu.VMEM_SHARED((npad, h), F32),
            pltpu.VMEM_SHARED((npad, h), F32),
            pltpu.SemaphoreType.DMA((NB,)),
            pltpu.SemaphoreType.DMA((NB,)),
        ],
    )
    def mp_kernel(y_hbm, src_hbm, dst_hbm, zrows_hbm, parts_hbm,
                  src_v, dst_v, bufs, accum, ysp, gsem, ssem):
        cid = lax.axis_index("c")
        sid = lax.axis_index("s")
        wid = cid * NS + sid
        base = sid * rows
        pltpu.sync_copy(zrows_hbm, accum.at[pl.ds(base, rows)])
        pltpu.sync_copy(y_hbm.at[pl.ds(base, rows)], ysp.at[pl.ds(base, rows)])
        pltpu.sync_copy(src_hbm.at[wid], src_v)
        pltpu.sync_copy(dst_hbm.at[wid], dst_v)
        plsc.subcore_barrier()

        def fire_gathers(g, b):
            for k in range(GRP):
                pltpu.async_copy(
                    ysp.at[src_v.at[g * GRP + k]],
                    bufs.at[b * GRP + k], gsem.at[b])

        def drain(sem, b):
            for k in range(GRP):
                pltpu.make_async_copy(
                    bufs.at[k], accum.at[pl.ds(0, CHUNK)], sem.at[b]).wait()

        fire_gathers(0, 0)

        def body(g, _):
            b = lax.rem(g, NB)
            nb = lax.rem(g + 1, NB)

            @pl.when(jnp.logical_and(g + 1 < ngrp, g + 1 >= NB))
            def _():
                drain(ssem, nb)

            @pl.when(g + 1 < ngrp)
            def _():
                fire_gathers(g + 1, nb)

            drain(gsem, b)
            for k in range(GRP):
                pltpu.async_copy(
                    bufs.at[b * GRP + k],
                    accum.at[dst_v.at[g * GRP + k]], ssem.at[b], add=True)
            return 0

        lax.fori_loop(0, ngrp, body, 0)
        for j in range(max(ngrp - NB, 0), ngrp):
            drain(ssem, j % NB)
        plsc.subcore_barrier()
        pltpu.sync_copy(accum.at[pl.ds(base, rows)],
                        parts_hbm.at[cid, pl.ds(base, rows)])

    return mp_kernel


def _tc1_body(x_ref, w_ref, deg_ref, y_ref, dinv_ref):
    d = deg_ref[0] + deg_ref[1] + 1.0
    dinv = lax.rsqrt(d)
    xw = jnp.dot(x_ref[...], w_ref[...], preferred_element_type=F32)
    y_ref[...] = xw * dinv
    dinv_ref[...] = dinv


def _tc2_body(p_ref, y1_ref, dinv_ref, b1_ref, w2_ref, y2_ref):
    dinv = dinv_ref[...]
    s = p_ref[0] + p_ref[1] + y1_ref[...]
    hh = jnp.maximum(dinv * s + b1_ref[...], 0.0)
    y2_ref[...] = dinv * jnp.dot(hh, w2_ref[...], preferred_element_type=F32)


def _tc3_body(p_ref, y2_ref, dinv_ref, b2_ref, o_ref):
    o = dinv_ref[...] * (p_ref[0] + p_ref[1] + y2_ref[...]) + b2_ref[...]
    m = jnp.max(o, axis=1, keepdims=True)
    e = jnp.exp(o - m)
    s = jnp.sum(e, axis=1, keepdims=True)
    o_ref[...] = o - m - jnp.log(s)


def kernel(x, edge_index, W1, b1, W2, b2):
    n, d_feat = x.shape
    e = edge_index.shape[1]
    h1 = W1.shape[1]
    h2 = W2.shape[1]

    npad = ((n + NS * 16) // (NS * 16)) * (NS * 16)
    nchunk = (-(-e // (NW * CHUNK * GRP))) * GRP
    epad = nchunk * NW * CHUNK
    epw = epad // NW

    pad = epad - e
    src_p = jnp.concatenate([edge_index[0], jnp.zeros((pad,), jnp.int32)])
    dst_p = jnp.concatenate(
        [edge_index[1], jnp.full((pad,), n, jnp.int32)])
    src_r = src_p.reshape(NW, nchunk, CHUNK)
    dst_r = dst_p.reshape(NW, nchunk, CHUNK)
    x_p = jnp.pad(x, ((0, npad - n), (0, 0)))
    zhist = jnp.zeros((npad,), F32)
    zrows = jnp.zeros((npad // NS, h1), F32)
    ones_c = jnp.ones((CHUNK,), F32)

    deg_kernel = _make_deg_kernel(npad, nchunk)
    mp1 = _make_mp_kernel(npad, h1, nchunk)

    degp = deg_kernel(dst_r, ones_c, zhist)

    y1, dinv = pl.pallas_call(
        _tc1_body,
        out_shape=(
            jax.ShapeDtypeStruct((npad, h1), F32),
            jax.ShapeDtypeStruct((npad, 1), F32),
        ),
    )(x_p, W1, degp.reshape(NC, npad, 1))

    p1 = mp1(y1, src_r, dst_r, zrows)

    y2 = pl.pallas_call(
        _tc2_body,
        out_shape=jax.ShapeDtypeStruct((npad, h2), F32),
    )(p1, y1, dinv, b1.reshape(1, h1), W2)

    if h2 != h1:
        mp2 = _make_mp_kernel(npad, h2, nchunk)
        zrows2 = jnp.zeros((npad // NS, h2), F32)
    else:
        mp2, zrows2 = mp1, zrows
    p2 = mp2(y2, src_r, dst_r, zrows2)

    out = pl.pallas_call(
        _tc3_body,
        out_shape=jax.ShapeDtypeStruct((npad, h2), F32),
    )(p2, y2, dinv, b2.reshape(1, h2))

    return out[:n]


# speedup vs baseline: 1.5346x; 1.1714x over previous
import functools

import jax
import jax.numpy as jnp
from jax import lax
from jax.experimental import pallas as pl
from jax.experimental.pallas import tpu as pltpu
from jax.experimental.pallas import tpu_sc as plsc

F32 = jnp.float32

NC = 2
NS = 16
NW = NC * NS
CHUNK = 128


def _mesh():
    return plsc.VectorSubcoreMesh(core_axis_name="c", subcore_axis_name="s")


def _make_deg_kernel(npad, nchunk):
    rows = npad // NS

    @functools.partial(
        pl.kernel,
        out_type=jax.ShapeDtypeStruct((NC, npad), F32),
        mesh=_mesh(),
        compiler_params=pltpu.CompilerParams(use_tc_tiling_on_sc=False),
        scratch_types=[
            pltpu.VMEM((nchunk, CHUNK), jnp.int32),
            pltpu.VMEM((CHUNK,), F32),
            pltpu.VMEM_SHARED((npad,), F32),
        ],
    )
    def deg_kernel(dst_hbm, ones_hbm, zhist_hbm, deg_hbm, idx_v, ones_v,
                   hist_sp):
        cid = lax.axis_index("c")
        sid = lax.axis_index("s")
        wid = cid * NS + sid
        base = sid * rows
        pltpu.sync_copy(zhist_hbm.at[pl.ds(base, rows)],
                        hist_sp.at[pl.ds(base, rows)])
        pltpu.sync_copy(dst_hbm.at[wid], idx_v)
        pltpu.sync_copy(ones_hbm, ones_v)
        plsc.subcore_barrier()

        def body(j, _):
            pltpu.sync_copy(ones_v, hist_sp.at[idx_v.at[j]], add=True)
            return 0

        lax.fori_loop(0, nchunk, body, 0)
        plsc.subcore_barrier()
        pltpu.sync_copy(hist_sp.at[pl.ds(base, rows)],
                        deg_hbm.at[cid, pl.ds(base, rows)])

    return deg_kernel


GRP = 4


def _rsqrt16(d):
    u = plsc.bitcast(d, jnp.int32)
    u = jnp.int32(0x5F3759DF) - lax.shift_right_logical(u, 1)
    z = plsc.bitcast(u, F32)
    for _ in range(3):
        z = z * (1.5 - 0.5 * d * z * z)
    return z


def _make_mp_kernel(npad, h, nchunk):
    rows = npad // NS
    ngrp = nchunk // GRP
    NB = 4

    @functools.partial(
        pl.kernel,
        out_type=jax.ShapeDtypeStruct((NC, npad, h), F32),
        mesh=_mesh(),
        compiler_params=pltpu.CompilerParams(
            use_tc_tiling_on_sc=False, needs_layout_passes=False),
        scratch_types=[
            pltpu.VMEM((nchunk, CHUNK), jnp.int32),
            pltpu.VMEM((nchunk, CHUNK), jnp.int32),
            pltpu.VMEM((NB * GRP, CHUNK, h), F32),
            pltpu.VMEM((rows, h), F32),
            pltpu.VMEM((rows,), F32),
            pltpu.VMEM((rows,), F32),
            pltpu.VMEM_SHARED((npad, h), F32),
            pltpu.VMEM_SHARED((npad, h), F32),
            pltpu.SemaphoreType.DMA((NB,)),
            pltpu.SemaphoreType.DMA((NB,)),
        ],
    )
    def mp_kernel(xw_hbm, degp_hbm, src_hbm, dst_hbm, zrows_hbm, parts_hbm,
                  src_v, dst_v, bufs, xl, dv, tv, accum, ysp, gsem, ssem):
        cid = lax.axis_index("c")
        sid = lax.axis_index("s")
        wid = cid * NS + sid
        base = sid * rows
        pltpu.sync_copy(xw_hbm.at[pl.ds(base, rows)], xl)
        pltpu.sync_copy(degp_hbm.at[0, pl.ds(base, rows)], dv)
        pltpu.sync_copy(degp_hbm.at[1, pl.ds(base, rows)], tv)
        pltpu.sync_copy(src_hbm.at[wid], src_v)
        pltpu.sync_copy(dst_hbm.at[wid], dst_v)

        def dinv_body(r, _):
            sl = pl.ds(r * 16, 16)
            dv[sl] = _rsqrt16(dv[sl] + tv[sl] + 1.0)
            return 0

        lax.fori_loop(0, rows // 16, dinv_body, 0)

        def scale_body(q, _):
            dd = dv[pl.ds(q * 16, 16)]
            for k in range(16):
                r = q * 16 + k
                xl[r, :] = xl[r, :] * dd[k]
            return 0

        lax.fori_loop(0, rows // 16, scale_body, 0)

        pltpu.sync_copy(xl, ysp.at[pl.ds(base, rows)])

        @pl.when(cid == 0)
        def _():
            pltpu.sync_copy(xl, accum.at[pl.ds(base, rows)])

        @pl.when(cid != 0)
        def _():
            pltpu.sync_copy(zrows_hbm, accum.at[pl.ds(base, rows)])

        plsc.subcore_barrier()

        def fire_gathers(g, b):
            for k in range(GRP):
                pltpu.async_copy(
                    ysp.at[src_v.at[g * GRP + k]],
                    bufs.at[b * GRP + k], gsem.at[b])

        def drain(sem, b):
            for k in range(GRP):
                pltpu.make_async_copy(
                    bufs.at[k], accum.at[pl.ds(0, CHUNK)], sem.at[b]).wait()

        fire_gathers(0, 0)

        def body(g, _):
            b = lax.rem(g, NB)
            nb = lax.rem(g + 1, NB)

            @pl.when(jnp.logical_and(g + 1 < ngrp, g + 1 >= NB))
            def _():
                drain(ssem, nb)

            @pl.when(g + 1 < ngrp)
            def _():
                fire_gathers(g + 1, nb)

            drain(gsem, b)
            for k in range(GRP):
                pltpu.async_copy(
                    bufs.at[b * GRP + k],
                    accum.at[dst_v.at[g * GRP + k]], ssem.at[b], add=True)
            return 0

        lax.fori_loop(0, ngrp, body, 0)
        for j in range(max(ngrp - NB, 0), ngrp):
            drain(ssem, j % NB)
        plsc.subcore_barrier()

        pltpu.sync_copy(accum.at[pl.ds(base, rows)], xl)

        def out_scale(q, _):
            dd = dv[pl.ds(q * 16, 16)]
            for k in range(16):
                r = q * 16 + k
                xl[r, :] = xl[r, :] * dd[k]
            return 0

        lax.fori_loop(0, rows // 16, out_scale, 0)
        pltpu.sync_copy(xl, parts_hbm.at[cid, pl.ds(base, rows)])

    return mp_kernel


def _tc1_body(x_ref, w_ref, y_ref):
    y_ref[...] = jnp.dot(x_ref[...], w_ref[...], preferred_element_type=F32)


def _tc2_body(p_ref, b1_ref, w2_ref, y2_ref):
    hh = jnp.maximum(p_ref[0] + p_ref[1] + b1_ref[...], 0.0)
    y2_ref[...] = jnp.dot(hh, w2_ref[...], preferred_element_type=F32)


def _tc3_body(p_ref, b2_ref, o_ref):
    o = p_ref[0] + p_ref[1] + b2_ref[...]
    m = jnp.max(o, axis=1, keepdims=True)
    e = jnp.exp(o - m)
    s = jnp.sum(e, axis=1, keepdims=True)
    o_ref[...] = o - m - jnp.log(s)


def kernel(x, edge_index, W1, b1, W2, b2):
    n, d_feat = x.shape
    e = edge_index.shape[1]
    h1 = W1.shape[1]
    h2 = W2.shape[1]

    npad = ((n + NS * 16) // (NS * 16)) * (NS * 16)
    nchunk = (-(-e // (NW * CHUNK * GRP))) * GRP
    epad = nchunk * NW * CHUNK
    epw = epad // NW

    pad = epad - e
    src_p = jnp.concatenate([edge_index[0], jnp.zeros((pad,), jnp.int32)])
    dst_p = jnp.concatenate(
        [edge_index[1], jnp.full((pad,), n, jnp.int32)])
    src_r = src_p.reshape(NW, nchunk, CHUNK)
    dst_r = dst_p.reshape(NW, nchunk, CHUNK)
    x_p = jnp.pad(x, ((0, npad - n), (0, 0)))
    zhist = jnp.zeros((npad,), F32)
    zrows = jnp.zeros((npad // NS, h1), F32)
    ones_c = jnp.ones((CHUNK,), F32)

    deg_kernel = _make_deg_kernel(npad, nchunk)
    mp1 = _make_mp_kernel(npad, h1, nchunk)

    degp = deg_kernel(dst_r, ones_c, zhist)

    xw1 = pl.pallas_call(
        _tc1_body,
        out_shape=jax.ShapeDtypeStruct((npad, h1), F32),
    )(x_p, W1)

    p1 = mp1(xw1, degp, src_r, dst_r, zrows)

    xw2 = pl.pallas_call(
        _tc2_body,
        out_shape=jax.ShapeDtypeStruct((npad, h2), F32),
    )(p1, b1.reshape(1, h1), W2)

    if h2 != h1:
        mp2 = _make_mp_kernel(npad, h2, nchunk)
        zrows2 = jnp.zeros((npad // NS, h2), F32)
    else:
        mp2, zrows2 = mp1, zrows
    p2 = mp2(xw2, degp, src_r, dst_r, zrows2)

    out = pl.pallas_call(
        _tc3_body,
        out_shape=jax.ShapeDtypeStruct((npad, h2), F32),
    )(p2, b2.reshape(1, h2))

    return out[:n]


# speedup vs baseline: 1.5774x; 1.0279x over previous
import functools

import jax
import jax.numpy as jnp
from jax import lax
from jax.experimental import pallas as pl
from jax.experimental.pallas import tpu as pltpu
from jax.experimental.pallas import tpu_sc as plsc

F32 = jnp.float32

NC = 2
NS = 16
NW = NC * NS
CHUNK = 128


def _mesh():
    return plsc.VectorSubcoreMesh(core_axis_name="c", subcore_axis_name="s")


def _make_deg_kernel(npad, nchunk):
    rows = npad // NS

    @functools.partial(
        pl.kernel,
        out_type=jax.ShapeDtypeStruct((NC, npad), F32),
        mesh=_mesh(),
        compiler_params=pltpu.CompilerParams(use_tc_tiling_on_sc=False),
        scratch_types=[
            pltpu.VMEM((nchunk, CHUNK), jnp.int32),
            pltpu.VMEM((CHUNK,), F32),
            pltpu.VMEM_SHARED((npad,), F32),
            pltpu.SemaphoreType.DMA,
        ],
    )
    def deg_kernel(dst_hbm, ones_hbm, zhist_hbm, deg_hbm, idx_v, ones_v,
                   hist_sp, sem):
        cid = lax.axis_index("c")
        sid = lax.axis_index("s")
        wid = cid * NS + sid
        base = sid * rows
        pltpu.sync_copy(zhist_hbm.at[pl.ds(base, rows)],
                        hist_sp.at[pl.ds(base, rows)])
        pltpu.sync_copy(dst_hbm.at[wid], idx_v)
        pltpu.sync_copy(ones_hbm, ones_v)
        plsc.subcore_barrier()

        DEPTH = 8

        def drain1():
            pltpu.make_async_copy(
                ones_v, hist_sp.at[pl.ds(0, CHUNK)], sem).wait()

        def body(j, _):
            @pl.when(j >= DEPTH)
            def _():
                drain1()

            pltpu.async_copy(ones_v, hist_sp.at[idx_v.at[j]], sem, add=True)
            return 0

        lax.fori_loop(0, nchunk, body, 0)

        def tail(j, _):
            drain1()
            return 0

        lax.fori_loop(0, DEPTH, tail, 0)
        plsc.subcore_barrier()
        pltpu.sync_copy(hist_sp.at[pl.ds(base, rows)],
                        deg_hbm.at[cid, pl.ds(base, rows)])

    return deg_kernel


GRP = 4


def _rsqrt16(d):
    u = plsc.bitcast(d, jnp.int32)
    u = jnp.int32(0x5F3759DF) - lax.shift_right_logical(u, 1)
    z = plsc.bitcast(u, F32)
    for _ in range(3):
        z = z * (1.5 - 0.5 * d * z * z)
    return z


def _make_mp_kernel(npad, h, nchunk):
    rows = npad // NS
    ngrp = nchunk // GRP
    NB = 4

    @functools.partial(
        pl.kernel,
        out_type=jax.ShapeDtypeStruct((NC, npad, h), F32),
        mesh=_mesh(),
        compiler_params=pltpu.CompilerParams(
            use_tc_tiling_on_sc=False, needs_layout_passes=False),
        scratch_types=[
            pltpu.VMEM((nchunk, CHUNK), jnp.int32),
            pltpu.VMEM((nchunk, CHUNK), jnp.int32),
            pltpu.VMEM((NB * GRP, CHUNK, h), F32),
            pltpu.VMEM((rows, h), F32),
            pltpu.VMEM((rows,), F32),
            pltpu.VMEM((rows,), F32),
            pltpu.VMEM_SHARED((npad, h), F32),
            pltpu.VMEM_SHARED((npad, h), F32),
            pltpu.SemaphoreType.DMA((NB,)),
            pltpu.SemaphoreType.DMA((NB,)),
        ],
    )
    def mp_kernel(xw_hbm, degp_hbm, src_hbm, dst_hbm, zrows_hbm, parts_hbm,
                  src_v, dst_v, bufs, xl, dv, tv, accum, ysp, gsem, ssem):
        cid = lax.axis_index("c")
        sid = lax.axis_index("s")
        wid = cid * NS + sid
        base = sid * rows
        pre = [
            pltpu.async_copy(xw_hbm.at[pl.ds(base, rows)], xl, gsem.at[0]),
            pltpu.async_copy(degp_hbm.at[0, pl.ds(base, rows)], dv,
                             gsem.at[0]),
            pltpu.async_copy(degp_hbm.at[1, pl.ds(base, rows)], tv,
                             gsem.at[0]),
            pltpu.async_copy(src_hbm.at[wid], src_v, gsem.at[1]),
            pltpu.async_copy(dst_hbm.at[wid], dst_v, gsem.at[1]),
        ]
        for d in pre:
            d.wait()

        def dinv_body(r, _):
            sl = pl.ds(r * 16, 16)
            dv[sl] = _rsqrt16(dv[sl] + tv[sl] + 1.0)
            return 0

        lax.fori_loop(0, rows // 16, dinv_body, 0)

        def scale_body(q, _):
            dd = dv[pl.ds(q * 16, 16)]
            for k in range(16):
                r = q * 16 + k
                xl[r, :] = xl[r, :] * dd[k]
            return 0

        lax.fori_loop(0, rows // 16, scale_body, 0)

        pltpu.sync_copy(xl, ysp.at[pl.ds(base, rows)])

        @pl.when(cid == 0)
        def _():
            pltpu.sync_copy(xl, accum.at[pl.ds(base, rows)])

        @pl.when(cid != 0)
        def _():
            pltpu.sync_copy(zrows_hbm, accum.at[pl.ds(base, rows)])

        plsc.subcore_barrier()

        def fire_gathers(g, b):
            for k in range(GRP):
                pltpu.async_copy(
                    ysp.at[src_v.at[g * GRP + k]],
                    bufs.at[b * GRP + k], gsem.at[b])

        def drain(sem, b):
            for k in range(GRP):
                pltpu.make_async_copy(
                    bufs.at[k], accum.at[pl.ds(0, CHUNK)], sem.at[b]).wait()

        fire_gathers(0, 0)

        def body(g, _):
            b = lax.rem(g, NB)
            nb = lax.rem(g + 1, NB)

            @pl.when(jnp.logical_and(g + 1 < ngrp, g + 1 >= NB))
            def _():
                drain(ssem, nb)

            @pl.when(g + 1 < ngrp)
            def _():
                fire_gathers(g + 1, nb)

            drain(gsem, b)
            for k in range(GRP):
                pltpu.async_copy(
                    bufs.at[b * GRP + k],
                    accum.at[dst_v.at[g * GRP + k]], ssem.at[b], add=True)
            return 0

        lax.fori_loop(0, ngrp, body, 0)
        for j in range(max(ngrp - NB, 0), ngrp):
            drain(ssem, j % NB)
        plsc.subcore_barrier()

        pltpu.sync_copy(accum.at[pl.ds(base, rows)], xl)

        def out_scale(q, _):
            dd = dv[pl.ds(q * 16, 16)]
            for k in range(16):
                r = q * 16 + k
                xl[r, :] = xl[r, :] * dd[k]
            return 0

        lax.fori_loop(0, rows // 16, out_scale, 0)
        pltpu.sync_copy(xl, parts_hbm.at[cid, pl.ds(base, rows)])

    return mp_kernel


def _tc1_body(x_ref, w_ref, y_ref):
    y_ref[...] = jnp.dot(x_ref[...], w_ref[...], preferred_element_type=F32)


def _tc2_body(p_ref, b1_ref, w2_ref, y2_ref):
    hh = jnp.maximum(p_ref[0] + p_ref[1] + b1_ref[...], 0.0)
    y2_ref[...] = jnp.dot(hh, w2_ref[...], preferred_element_type=F32)


def _tc3_body(p_ref, b2_ref, o_ref):
    o = p_ref[0] + p_ref[1] + b2_ref[...]
    m = jnp.max(o, axis=1, keepdims=True)
    e = jnp.exp(o - m)
    s = jnp.sum(e, axis=1, keepdims=True)
    o_ref[...] = o - m - jnp.log(s)


def kernel(x, edge_index, W1, b1, W2, b2):
    n, d_feat = x.shape
    e = edge_index.shape[1]
    h1 = W1.shape[1]
    h2 = W2.shape[1]

    npad = ((n + NS * 16) // (NS * 16)) * (NS * 16)
    nchunk = (-(-e // (NW * CHUNK * GRP))) * GRP
    epad = nchunk * NW * CHUNK
    epw = epad // NW

    pad = epad - e
    src_p = jnp.concatenate([edge_index[0], jnp.zeros((pad,), jnp.int32)])
    dst_p = jnp.concatenate(
        [edge_index[1], jnp.full((pad,), n, jnp.int32)])
    src_r = src_p.reshape(NW, nchunk, CHUNK)
    dst_r = dst_p.reshape(NW, nchunk, CHUNK)
    x_p = jnp.pad(x, ((0, npad - n), (0, 0)))
    zhist = jnp.zeros((npad,), F32)
    zrows = jnp.zeros((npad // NS, h1), F32)
    ones_c = jnp.ones((CHUNK,), F32)

    deg_kernel = _make_deg_kernel(npad, nchunk)
    mp1 = _make_mp_kernel(npad, h1, nchunk)

    degp = deg_kernel(dst_r, ones_c, zhist)

    xw1 = pl.pallas_call(
        _tc1_body,
        out_shape=jax.ShapeDtypeStruct((npad, h1), F32),
    )(x_p, W1)

    p1 = mp1(xw1, degp, src_r, dst_r, zrows)

    xw2 = pl.pallas_call(
        _tc2_body,
        out_shape=jax.ShapeDtypeStruct((npad, h2), F32),
    )(p1, b1.reshape(1, h1), W2)

    if h2 != h1:
        mp2 = _make_mp_kernel(npad, h2, nchunk)
        zrows2 = jnp.zeros((npad // NS, h2), F32)
    else:
        mp2, zrows2 = mp1, zrows
    p2 = mp2(xw2, degp, src_r, dst_r, zrows2)

    out = pl.pallas_call(
        _tc3_body,
        out_shape=jax.ShapeDtypeStruct((npad, h2), F32),
    )(p2, b2.reshape(1, h2))

    return out[:n]


# speedup vs baseline: 1.8876x; 1.1966x over previous
import functools

import jax
import jax.numpy as jnp
from jax import lax
from jax.experimental import pallas as pl
from jax.experimental.pallas import tpu as pltpu
from jax.experimental.pallas import tpu_sc as plsc

F32 = jnp.float32

NC = 2
NS = 16
NW = NC * NS
CHUNK = 128


def _mesh():
    return plsc.VectorSubcoreMesh(core_axis_name="c", subcore_axis_name="s")


def _make_deg_kernel(npad, nchunk):
    rows = npad // NS

    @functools.partial(
        pl.kernel,
        out_type=jax.ShapeDtypeStruct((NC, npad), F32),
        mesh=_mesh(),
        compiler_params=pltpu.CompilerParams(use_tc_tiling_on_sc=False),
        scratch_types=[
            pltpu.VMEM((nchunk, CHUNK), jnp.int32),
            pltpu.VMEM((CHUNK,), F32),
            pltpu.VMEM_SHARED((npad,), F32),
            pltpu.SemaphoreType.DMA,
        ],
    )
    def deg_kernel(dst_hbm, ones_hbm, zhist_hbm, deg_hbm, idx_v, ones_v,
                   hist_sp, sem):
        cid = lax.axis_index("c")
        sid = lax.axis_index("s")
        wid = cid * NS + sid
        base = sid * rows
        pltpu.sync_copy(zhist_hbm.at[pl.ds(base, rows)],
                        hist_sp.at[pl.ds(base, rows)])
        pltpu.sync_copy(dst_hbm.at[wid], idx_v)
        pltpu.sync_copy(ones_hbm, ones_v)
        plsc.subcore_barrier()

        DEPTH = 8

        def drain1():
            pltpu.make_async_copy(
                ones_v, hist_sp.at[pl.ds(0, CHUNK)], sem).wait()

        def body(j, _):
            @pl.when(j >= DEPTH)
            def _():
                drain1()

            pltpu.async_copy(ones_v, hist_sp.at[idx_v.at[j]], sem, add=True)
            return 0

        lax.fori_loop(0, nchunk, body, 0)

        def tail(j, _):
            drain1()
            return 0

        lax.fori_loop(0, DEPTH, tail, 0)
        plsc.subcore_barrier()
        pltpu.sync_copy(hist_sp.at[pl.ds(base, rows)],
                        deg_hbm.at[cid, pl.ds(base, rows)])

    return deg_kernel


GRP = 4


def _rsqrt16(d):
    u = plsc.bitcast(d, jnp.int32)
    u = jnp.int32(0x5F3759DF) - lax.shift_right_logical(u, 1)
    z = plsc.bitcast(u, F32)
    for _ in range(3):
        z = z * (1.5 - 0.5 * d * z * z)
    return z


def _make_mp_kernel(npad, h, nchunk):
    rows = npad // NS
    ngrp = nchunk // GRP
    NB = 4

    @functools.partial(
        pl.kernel,
        out_type=jax.ShapeDtypeStruct((NC, npad, h), F32),
        mesh=_mesh(),
        compiler_params=pltpu.CompilerParams(
            use_tc_tiling_on_sc=False, needs_layout_passes=False),
        scratch_types=[
            pltpu.VMEM((nchunk, CHUNK), jnp.int32),
            pltpu.VMEM((nchunk, CHUNK), jnp.int32),
            pltpu.VMEM((NB * GRP, CHUNK, h), F32),
            pltpu.VMEM((rows, h), F32),
            pltpu.VMEM((rows,), F32),
            pltpu.VMEM((rows,), F32),
            pltpu.VMEM_SHARED((npad, h), F32),
            pltpu.VMEM_SHARED((npad, h), F32),
            pltpu.SemaphoreType.DMA((NB,)),
            pltpu.SemaphoreType.DMA((NB,)),
        ],
    )
    def mp_kernel(xw_hbm, degp_hbm, src_hbm, dst_hbm, zrows_hbm, parts_hbm,
                  src_v, dst_v, bufs, xl, dv, tv, accum, ysp, gsem, ssem):
        cid = lax.axis_index("c")
        sid = lax.axis_index("s")
        wid = cid * NS + sid
        base = sid * rows
        pre = [
            pltpu.async_copy(xw_hbm.at[pl.ds(base, rows)], xl, gsem.at[0]),
            pltpu.async_copy(degp_hbm.at[0, pl.ds(base, rows)], dv,
                             gsem.at[0]),
            pltpu.async_copy(degp_hbm.at[1, pl.ds(base, rows)], tv,
                             gsem.at[0]),
            pltpu.async_copy(src_hbm.at[wid], src_v, gsem.at[1]),
            pltpu.async_copy(dst_hbm.at[wid], dst_v, gsem.at[1]),
        ]
        for d in pre:
            d.wait()

        def dinv_body(r, _):
            sl = pl.ds(r * 16, 16)
            dv[sl] = _rsqrt16(dv[sl] + tv[sl] + 1.0)
            return 0

        lax.fori_loop(0, rows // 16, dinv_body, 0)

        def scale_body(q, _):
            dd = dv[pl.ds(q * 16, 16)]
            for k in range(16):
                r = q * 16 + k
                xl[r, :] = xl[r, :] * dd[k]
            return 0

        lax.fori_loop(0, rows // 16, scale_body, 0)

        pltpu.sync_copy(xl, ysp.at[pl.ds(base, rows)])

        @pl.when(cid == 0)
        def _():
            pltpu.sync_copy(xl, accum.at[pl.ds(base, rows)])

        @pl.when(cid != 0)
        def _():
            pltpu.sync_copy(zrows_hbm, accum.at[pl.ds(base, rows)])

        plsc.subcore_barrier()

        def fire_gathers(g, b):
            for k in range(GRP):
                pltpu.async_copy(
                    ysp.at[src_v.at[g * GRP + k]],
                    bufs.at[b * GRP + k], gsem.at[b])

        def drain(sem, b):
            for k in range(GRP):
                pltpu.make_async_copy(
                    bufs.at[k], accum.at[pl.ds(0, CHUNK)], sem.at[b]).wait()

        fire_gathers(0, 0)

        def body(g, _):
            b = lax.rem(g, NB)
            nb = lax.rem(g + 1, NB)

            @pl.when(jnp.logical_and(g + 1 < ngrp, g + 1 >= NB))
            def _():
                drain(ssem, nb)

            @pl.when(g + 1 < ngrp)
            def _():
                fire_gathers(g + 1, nb)

            drain(gsem, b)
            for k in range(GRP):
                pltpu.async_copy(
                    bufs.at[b * GRP + k],
                    accum.at[dst_v.at[g * GRP + k]], ssem.at[b], add=True)
            return 0

        lax.fori_loop(0, ngrp, body, 0)
        for j in range(max(ngrp - NB, 0), ngrp):
            drain(ssem, j % NB)
        plsc.subcore_barrier()

        pltpu.sync_copy(accum.at[pl.ds(base, rows)], xl)

        def out_scale(q, _):
            dd = dv[pl.ds(q * 16, 16)]
            for k in range(16):
                r = q * 16 + k
                xl[r, :] = xl[r, :] * dd[k]
            return 0

        lax.fori_loop(0, rows // 16, out_scale, 0)
        pltpu.sync_copy(xl, parts_hbm.at[cid, pl.ds(base, rows)])

    return mp_kernel


def _tc1_body(x_ref, w_ref, y_ref):
    y_ref[...] = jnp.dot(x_ref[...], w_ref[...], preferred_element_type=F32)


def _tc2_body(p_ref, b1_ref, w2_ref, y2_ref):
    hh = jnp.maximum(p_ref[0] + p_ref[1] + b1_ref[...], 0.0)
    y2_ref[...] = jnp.dot(hh, w2_ref[...], preferred_element_type=F32)


def _make_tc3_body(h):
    ng = 128 // h

    def _tc3_body(p_ref, b2_ref, o_ref):
        o = p_ref[0] + p_ref[1] + b2_ref[...]
        m = jnp.concatenate(
            [jnp.broadcast_to(
                jnp.max(o[:, j * h:(j + 1) * h], axis=1, keepdims=True),
                (o.shape[0], h)) for j in range(ng)], axis=1)
        e = jnp.exp(o - m)
        s = jnp.concatenate(
            [jnp.broadcast_to(
                jnp.sum(e[:, j * h:(j + 1) * h], axis=1, keepdims=True),
                (o.shape[0], h)) for j in range(ng)], axis=1)
        o_ref[...] = o - m - jnp.log(s)

    return _tc3_body


def kernel(x, edge_index, W1, b1, W2, b2):
    n, d_feat = x.shape
    e = edge_index.shape[1]
    h1 = W1.shape[1]
    h2 = W2.shape[1]

    npad = ((n + NS * 16) // (NS * 16)) * (NS * 16)
    nchunk = (-(-e // (NW * CHUNK * GRP))) * GRP
    epad = nchunk * NW * CHUNK
    epw = epad // NW

    pad = epad - e
    src_p = jnp.concatenate([edge_index[0], jnp.zeros((pad,), jnp.int32)])
    dst_p = jnp.concatenate(
        [edge_index[1], jnp.full((pad,), n, jnp.int32)])
    src_r = src_p.reshape(NW, nchunk, CHUNK)
    dst_r = dst_p.reshape(NW, nchunk, CHUNK)
    x_p = jnp.pad(x, ((0, npad - n), (0, 0)))
    zhist = jnp.zeros((npad,), F32)
    zrows = jnp.zeros((npad // NS, h1), F32)
    ones_c = jnp.ones((CHUNK,), F32)

    deg_kernel = _make_deg_kernel(npad, nchunk)
    mp1 = _make_mp_kernel(npad, h1, nchunk)

    degp = deg_kernel(dst_r, ones_c, zhist)

    pk1 = 128 // h1
    pk2 = 128 // h2
    eye1 = jnp.eye(pk1, dtype=F32)
    eye2 = jnp.eye(pk2, dtype=F32)

    xw1 = pl.pallas_call(
        _tc1_body,
        out_shape=jax.ShapeDtypeStruct((npad // pk1, pk1 * h1), F32),
    )(x_p.reshape(npad // pk1, pk1 * d_feat), jnp.kron(eye1, W1))

    p1 = mp1(xw1.reshape(npad, h1), degp, src_r, dst_r, zrows)

    xw2 = pl.pallas_call(
        _tc2_body,
        out_shape=jax.ShapeDtypeStruct((npad // pk2, pk2 * h2), F32),
    )(p1.reshape(NC, npad // pk1, pk1 * h1),
      jnp.tile(b1, pk1).reshape(1, pk1 * h1), jnp.kron(eye2, W2))

    if h2 != h1:
        mp2 = _make_mp_kernel(npad, h2, nchunk)
        zrows2 = jnp.zeros((npad // NS, h2), F32)
    else:
        mp2, zrows2 = mp1, zrows
    p2 = mp2(xw2.reshape(npad, h2), degp, src_r, dst_r, zrows2)

    out = pl.pallas_call(
        _make_tc3_body(h2),
        out_shape=jax.ShapeDtypeStruct((npad // pk2, pk2 * h2), F32),
    )(p2.reshape(NC, npad // pk2, pk2 * h2),
      jnp.tile(b2, pk2).reshape(1, pk2 * h2))

    return out.reshape(npad, h2)[:n]


# speedup vs baseline: 1.9694x; 1.0433x over previous
import functools

import jax
import jax.numpy as jnp
from jax import lax
from jax.experimental import pallas as pl
from jax.experimental.pallas import tpu as pltpu
from jax.experimental.pallas import tpu_sc as plsc

F32 = jnp.float32

NC = 2
NS = 16
NW = NC * NS
GRP = 4


def _pick_chunk(e):
    if e % NW == 0:
        per = e // NW
        for c in range(128, 0, -1):
            if per % c == 0 and (per // c) % GRP == 0:
                return c, per // c, 0
    c = 128
    nchunk = (-(-e // (NW * c * GRP))) * GRP
    return c, nchunk, NW * c * nchunk - e


def _mesh():
    return plsc.VectorSubcoreMesh(core_axis_name="c", subcore_axis_name="s")


def _make_deg_kernel(npad, nchunk, ck):
    rows = npad // NS
    ckp = -(-ck // 16) * 16

    @functools.partial(
        pl.kernel,
        out_type=jax.ShapeDtypeStruct((NC, npad), F32),
        mesh=_mesh(),
        compiler_params=pltpu.CompilerParams(use_tc_tiling_on_sc=False),
        scratch_types=[
            pltpu.VMEM((nchunk, ck), jnp.int32),
            pltpu.VMEM((ckp,), F32),
            pltpu.VMEM((rows,), F32),
            pltpu.VMEM_SHARED((npad,), F32),
            pltpu.SemaphoreType.DMA,
            pltpu.SemaphoreType.DMA,
        ],
    )
    def deg_kernel(dst_hbm, deg_hbm, idx_v, ones_v, zv, hist_sp, sem, zsem):
        cid = lax.axis_index("c")
        sid = lax.axis_index("s")
        wid = cid * NS + sid
        base = sid * rows
        dload = pltpu.async_copy(dst_hbm.at[wid], idx_v, zsem)

        def fill_ones(i, _):
            ones_v[pl.ds(i * 16, 16)] = jnp.full((16,), 1.0, F32)
            return 0

        lax.fori_loop(0, ckp // 16, fill_ones, 0)

        def fill_zero(i, _):
            zv[pl.ds(i * 16, 16)] = jnp.zeros((16,), F32)
            return 0

        lax.fori_loop(0, rows // 16, fill_zero, 0)
        pltpu.sync_copy(zv, hist_sp.at[pl.ds(base, rows)])
        dload.wait()
        plsc.subcore_barrier()

        DEPTH = 8

        def drain1():
            pltpu.make_async_copy(
                ones_v.at[pl.ds(0, ck)], hist_sp.at[pl.ds(0, ck)],
                sem).wait()

        def body(j, _):
            @pl.when(j >= DEPTH)
            def _():
                drain1()

            pltpu.async_copy(ones_v.at[pl.ds(0, ck)], hist_sp.at[idx_v.at[j]],
                             sem, add=True)
            return 0

        lax.fori_loop(0, nchunk, body, 0)

        def tail(j, _):
            drain1()
            return 0

        lax.fori_loop(0, DEPTH, tail, 0)
        plsc.subcore_barrier()
        pltpu.sync_copy(hist_sp.at[pl.ds(base, rows)],
                        deg_hbm.at[cid, pl.ds(base, rows)])

    return deg_kernel


def _rsqrt16(d):
    u = plsc.bitcast(d, jnp.int32)
    u = jnp.int32(0x5F3759DF) - lax.shift_right_logical(u, 1)
    z = plsc.bitcast(u, F32)
    for _ in range(3):
        z = z * (1.5 - 0.5 * d * z * z)
    return z


def _make_mp_kernel(npad, h, nchunk, ck):
    rows = npad // NS
    ngrp = nchunk // GRP
    NB = 4

    @functools.partial(
        pl.kernel,
        out_type=jax.ShapeDtypeStruct((NC, npad, h), F32),
        mesh=_mesh(),
        compiler_params=pltpu.CompilerParams(
            use_tc_tiling_on_sc=False, needs_layout_passes=False),
        scratch_types=[
            pltpu.VMEM((nchunk, ck), jnp.int32),
            pltpu.VMEM((nchunk, ck), jnp.int32),
            pltpu.VMEM((NB * GRP, ck, h), F32),
            pltpu.VMEM((rows, h), F32),
            pltpu.VMEM((rows, h), F32),
            pltpu.VMEM((rows,), F32),
            pltpu.VMEM((rows,), F32),
            pltpu.VMEM_SHARED((npad, h), F32),
            pltpu.VMEM_SHARED((npad, h), F32),
            pltpu.SemaphoreType.DMA((NB,)),
            pltpu.SemaphoreType.DMA((NB,)),
        ],
    )
    def mp_kernel(xw_hbm, degp_hbm, src_hbm, dst_hbm, parts_hbm,
                  src_v, dst_v, bufs, xl, yh, dv, tv, accum, ysp, gsem, ssem):
        cid = lax.axis_index("c")
        sid = lax.axis_index("s")
        wid = cid * NS + sid
        base = sid * rows
        pre = [
            pltpu.async_copy(xw_hbm.at[pl.ds(base, rows)], xl, gsem.at[0]),
            pltpu.async_copy(degp_hbm.at[0, pl.ds(base, rows)], dv,
                             gsem.at[0]),
            pltpu.async_copy(degp_hbm.at[1, pl.ds(base, rows)], tv,
                             gsem.at[0]),
            pltpu.async_copy(src_hbm.at[wid], src_v, gsem.at[1]),
            pltpu.async_copy(dst_hbm.at[wid], dst_v, gsem.at[1]),
        ]
        for d in pre:
            d.wait()

        def dinv_body(r, _):
            sl = pl.ds(r * 16, 16)
            dv[sl] = _rsqrt16(dv[sl] + tv[sl] + 1.0)
            return 0

        lax.fori_loop(0, rows // 16, dinv_body, 0)

        def scale_body(q, _):
            dd = dv[pl.ds(q * 16, 16)]
            for k in range(16):
                r = q * 16 + k
                v = xl[r, :] * dd[k]
                xl[r, :] = v
                yh[r, :] = v * 0.5
            return 0

        lax.fori_loop(0, rows // 16, scale_body, 0)

        pltpu.sync_copy(xl, ysp.at[pl.ds(base, rows)])
        pltpu.sync_copy(yh, accum.at[pl.ds(base, rows)])
        plsc.subcore_barrier()

        def fire_gathers(g, b):
            for k in range(GRP):
                pltpu.async_copy(
                    ysp.at[src_v.at[g * GRP + k]],
                    bufs.at[b * GRP + k], gsem.at[b])

        def drain(sem, b):
            for k in range(GRP):
                pltpu.make_async_copy(
                    bufs.at[k], accum.at[pl.ds(0, ck)], sem.at[b]).wait()

        fire_gathers(0, 0)

        def body(g, _):
            b = lax.rem(g, NB)
            nb = lax.rem(g + 1, NB)

            @pl.when(jnp.logical_and(g + 1 < ngrp, g + 1 >= NB))
            def _():
                drain(ssem, nb)

            @pl.when(g + 1 < ngrp)
            def _():
                fire_gathers(g + 1, nb)

            drain(gsem, b)
            for k in range(GRP):
                pltpu.async_copy(
                    bufs.at[b * GRP + k],
                    accum.at[dst_v.at[g * GRP + k]], ssem.at[b], add=True)
            return 0

        lax.fori_loop(0, ngrp, body, 0)
        for j in range(max(ngrp - NB, 0), ngrp):
            drain(ssem, j % NB)
        plsc.subcore_barrier()

        pltpu.sync_copy(accum.at[pl.ds(base, rows)], xl)

        def out_scale(q, _):
            dd = dv[pl.ds(q * 16, 16)]
            for k in range(16):
                r = q * 16 + k
                xl[r, :] = xl[r, :] * dd[k]
            return 0

        lax.fori_loop(0, rows // 16, out_scale, 0)
        pltpu.sync_copy(xl, parts_hbm.at[cid, pl.ds(base, rows)])

    return mp_kernel


def _tc1_body(x_ref, w_ref, y_ref):
    y_ref[...] = jnp.dot(x_ref[...], w_ref[...], preferred_element_type=F32)


def _tc2_body(p_ref, b1_ref, w2_ref, y2_ref):
    hh = jnp.maximum(p_ref[0] + p_ref[1] + b1_ref[...], 0.0)
    y2_ref[...] = jnp.dot(hh, w2_ref[...], preferred_element_type=F32)


def _make_tc3_body(h, orows):
    ng = 128 // h

    def _tc3_body(p_ref, b2_ref, o_ref):
        o = (p_ref[0] + p_ref[1])[:orows] + b2_ref[...]
        m = jnp.concatenate(
            [jnp.broadcast_to(
                jnp.max(o[:, j * h:(j + 1) * h], axis=1, keepdims=True),
                (o.shape[0], h)) for j in range(ng)], axis=1)
        e = jnp.exp(o - m)
        s = jnp.concatenate(
            [jnp.broadcast_to(
                jnp.sum(e[:, j * h:(j + 1) * h], axis=1, keepdims=True),
                (o.shape[0], h)) for j in range(ng)], axis=1)
        o_ref[...] = o - m - jnp.log(s)

    return _tc3_body


def kernel(x, edge_index, W1, b1, W2, b2):
    n, d_feat = x.shape
    e = edge_index.shape[1]
    h1 = W1.shape[1]
    h2 = W2.shape[1]

    npad = ((n + NS * 16) // (NS * 16)) * (NS * 16)
    ck, nchunk, pad = _pick_chunk(e)

    if pad:
        src_p = jnp.concatenate([edge_index[0], jnp.zeros((pad,), jnp.int32)])
        dst_p = jnp.concatenate(
            [edge_index[1], jnp.full((pad,), n, jnp.int32)])
        src_r = src_p.reshape(NW, nchunk, ck)
        dst_r = dst_p.reshape(NW, nchunk, ck)
    else:
        er = edge_index.reshape(2, NW, nchunk, ck)
        src_r = er[0]
        dst_r = er[1]

    deg_kernel = _make_deg_kernel(npad, nchunk, ck)
    mp1 = _make_mp_kernel(npad, h1, nchunk, ck)

    degp = deg_kernel(dst_r)

    pk1 = 128 // h1
    pk2 = 128 // h2
    eye1 = jnp.eye(pk1, dtype=F32)
    eye2 = jnp.eye(pk2, dtype=F32)

    if n % pk1 == 0:
        x_pk = jnp.pad(x.reshape(n // pk1, pk1 * d_feat),
                       ((0, (npad - n) // pk1), (0, 0)))
    else:
        x_pk = jnp.pad(x, ((0, npad - n), (0, 0))).reshape(
            npad // pk1, pk1 * d_feat)

    xw1 = pl.pallas_call(
        _tc1_body,
        out_shape=jax.ShapeDtypeStruct((npad // pk1, pk1 * h1), F32),
    )(x_pk, jnp.kron(eye1, W1))

    p1 = mp1(xw1.reshape(npad, h1), degp, src_r, dst_r)

    xw2 = pl.pallas_call(
        _tc2_body,
        out_shape=jax.ShapeDtypeStruct((npad // pk2, pk2 * h2), F32),
    )(p1.reshape(NC, npad // pk1, pk1 * h1),
      jnp.tile(b1, pk1).reshape(1, pk1 * h1), jnp.kron(eye2, W2))

    mp2 = _make_mp_kernel(npad, h2, nchunk, ck) if h2 != h1 else mp1
    p2 = mp2(xw2.reshape(npad, h2), degp, src_r, dst_r)

    orows = n // pk2 if n % pk2 == 0 else npad // pk2
    out = pl.pallas_call(
        _make_tc3_body(h2, orows),
        out_shape=jax.ShapeDtypeStruct((orows, pk2 * h2), F32),
    )(p2.reshape(NC, npad // pk2, pk2 * h2),
      jnp.tile(b2, pk2).reshape(1, pk2 * h2))

    return out.reshape(orows * pk2, h2)[:n]
